# baseline (speedup 1.0000x reference)
import jax
import jax.numpy as jnp
from jax import lax
from jax.experimental import pallas as pl
from jax.experimental.pallas import tpu as pltpu
from jax.experimental.pallas import tpu_sc as plsc

_B = 16384
_D = 32
_PK = 4
_VW = _PK * _D
_NC = 2
_NS = 16
_NW = _NC * _NS
_BPW = _B // _NW
_CH = 128
_NCH = _BPW // _CH


def _sc_gather_body(uidx_hbm, iidx_hbm, ut_hbm, it_hbm, uout_hbm, iout_hbm,
                    uidx_v, iidx_v, urows_v, irows_v, su, si):
    c = lax.axis_index("c")
    s = lax.axis_index("s")
    wid = s * _NC + c
    base = wid * _BPW

    pltpu.sync_copy(uidx_hbm.at[pl.ds(base, _BPW)], uidx_v)
    pltpu.sync_copy(iidx_hbm.at[pl.ds(base, _BPW)], iidx_v)

    for j in range(_BPW // 16):
        sl = pl.ds(j * 16, 16)
        uidx_v[sl] = lax.shift_right_logical(uidx_v[sl], 2)
        iidx_v[sl] = lax.shift_right_logical(iidx_v[sl], 2)

    for k in range(_NCH):
        sl = pl.ds(k * _CH, _CH)
        cu = pltpu.async_copy(ut_hbm.at[uidx_v.at[sl]], urows_v, su)
        ci = pltpu.async_copy(it_hbm.at[iidx_v.at[sl]], irows_v, si)
        cu.wait()
        ci.wait()
        pltpu.sync_copy(urows_v, uout_hbm.at[pl.ds(base + k * _CH, _CH)])
        pltpu.sync_copy(irows_v, iout_hbm.at[pl.ds(base + k * _CH, _CH)])


_sc_gather = pl.kernel(
    _sc_gather_body,
    out_type=(jax.ShapeDtypeStruct((_B, _VW), jnp.float32),
              jax.ShapeDtypeStruct((_B, _VW), jnp.float32)),
    mesh=plsc.VectorSubcoreMesh(core_axis_name="c", subcore_axis_name="s"),
    scratch_types=[
        pltpu.VMEM((_BPW,), jnp.int32),
        pltpu.VMEM((_BPW,), jnp.int32),
        pltpu.VMEM((_CH, _VW), jnp.float32),
        pltpu.VMEM((_CH, _VW), jnp.float32),
        pltpu.SemaphoreType.DMA,
        pltpu.SemaphoreType.DMA,
    ],
    name="sc_gather_rows",
)


_TB = 2048


def _tc_body(urows_ref, irows_ref, mu_ref, mi_ref, wt_ref, bt_ref, wo_ref,
             bo_ref, out_ref):
    u = mu_ref[:, 0:1] * urows_ref[:, 0:_D]
    it = mi_ref[:, 0:1] * irows_ref[:, 0:_D]
    for k in range(1, _PK):
        u = u + mu_ref[:, k:k + 1] * urows_ref[:, k * _D:(k + 1) * _D]
        it = it + mi_ref[:, k:k + 1] * irows_ref[:, k * _D:(k + 1) * _D]
    p = u * it
    h = jnp.dot(p, wt_ref[...], preferred_element_type=jnp.float32)
    h = jnp.maximum(h + bt_ref[...], 0.0)
    out_ref[...] = jnp.dot(h, wo_ref[...],
                           preferred_element_type=jnp.float32) + bo_ref[0, 0]


_tc_mlp = pl.pallas_call(
    _tc_body,
    grid=(_B // _TB,),
    in_specs=[
        pl.BlockSpec((_TB, _VW), lambda i: (i, 0)),
        pl.BlockSpec((_TB, _VW), lambda i: (i, 0)),
        pl.BlockSpec((_TB, _PK), lambda i: (i, 0)),
        pl.BlockSpec((_TB, _PK), lambda i: (i, 0)),
        pl.BlockSpec((_D, _D), lambda i: (0, 0)),
        pl.BlockSpec((1, _D), lambda i: (0, 0)),
        pl.BlockSpec((_D, 1), lambda i: (0, 0)),
        pl.BlockSpec((1, 1), lambda i: (0, 0)),
    ],
    out_specs=pl.BlockSpec((_TB, 1), lambda i: (i, 0)),
    out_shape=jax.ShapeDtypeStruct((_B, 1), jnp.float32),
    name="tc_mlp",
)


def kernel(user_idx, item_idx, user_table, item_table, W_t, b_t, W_o, b_o):
    uidx = user_idx.astype(jnp.int32)
    iidx = item_idx.astype(jnp.int32)
    urows, irows = _sc_gather(uidx, iidx,
                              user_table.reshape(-1, _VW),
                              item_table.reshape(-1, _VW))
    mu = jax.nn.one_hot(uidx & (_PK - 1), _PK, dtype=jnp.float32)
    mi = jax.nn.one_hot(iidx & (_PK - 1), _PK, dtype=jnp.float32)
    pred = _tc_mlp(urows, irows, mu, mi, W_t, b_t.reshape(1, _D), W_o,
                   b_o.reshape(1, 1))
    return pred.reshape(_B)


# speedup vs baseline: 4.0485x; 4.0485x over previous
import functools

import jax
import jax.numpy as jnp
from jax import lax
from jax.experimental import pallas as pl
from jax.experimental.pallas import tpu as pltpu
from jax.experimental.pallas import tpu_sc as plsc

_B = 16384
_D = 32
_NC = 2
_NS = 16
_NW = _NC * _NS
_BPW = _B // _NW
_NBUF = 8


def _scal(ref, pos):
    return ref[pl.ds(pos, 16)][0]


def _sc_body(uidx_hbm, iidx_hbm, utT_hbm, itT_hbm, out_hbm,
             uidx_v, iidx_v, prod_v, bufs_and_sems):
    c = lax.axis_index("c")
    s = lax.axis_index("s")
    wid = s * _NC + c
    base = wid * _BPW

    ubufs = bufs_and_sems[:_NBUF]
    ibufs = bufs_and_sems[_NBUF:2 * _NBUF]
    usems = bufs_and_sems[2 * _NBUF:3 * _NBUF]
    isems = bufs_and_sems[3 * _NBUF:]

    pltpu.sync_copy(uidx_hbm.at[pl.ds(base, _BPW)], uidx_v.at[pl.ds(0, _BPW)])
    pltpu.sync_copy(iidx_hbm.at[pl.ds(base, _BPW)], iidx_v.at[pl.ds(0, _BPW)])

    def issue(n, b):
        tu = (lax.shift_right_logical(_scal(uidx_v, n), 7)) * 128
        ti = (lax.shift_right_logical(_scal(iidx_v, n), 7)) * 128
        pltpu.async_copy(utT_hbm.at[:, pl.ds(pl.multiple_of(tu, 128), 128)],
                         ubufs[b], usems[b])
        pltpu.async_copy(itT_hbm.at[:, pl.ds(pl.multiple_of(ti, 128), 128)],
                         ibufs[b], isems[b])

    rows_lo = lax.iota(jnp.int32, 16)
    rows_hi = rows_lo + 16

    def consume(n, b):
        pltpu.make_async_copy(utT_hbm.at[:, pl.ds(0, 128)], ubufs[b],
                              usems[b]).wait()
        pltpu.make_async_copy(itT_hbm.at[:, pl.ds(0, 128)], ibufs[b],
                              isems[b]).wait()
        lu = _scal(uidx_v, n) & 127
        li = _scal(iidx_v, n) & 127
        cu = jnp.full((16,), lu, dtype=jnp.int32)
        ci = jnp.full((16,), li, dtype=jnp.int32)
        u0 = plsc.load_gather(ubufs[b], [rows_lo, cu])
        u1 = plsc.load_gather(ubufs[b], [rows_hi, cu])
        i0 = plsc.load_gather(ibufs[b], [rows_lo, ci])
        i1 = plsc.load_gather(ibufs[b], [rows_hi, ci])
        prod_v[pl.ds(n * _D, 16)] = u0 * i0
        prod_v[pl.ds(n * _D + 16, 16)] = u1 * i1

    for b in range(_NBUF):
        issue(b, b)

    def ring(i, carry):
        for b in range(_NBUF):
            n = i * _NBUF + b
            consume(n, b)
            issue(n + _NBUF, b)
        return carry

    lax.fori_loop(0, _BPW // _NBUF - 1, ring, 0)
    for b in range(_NBUF):
        consume(_BPW - _NBUF + b, b)

    pltpu.sync_copy(prod_v, out_hbm.at[pl.ds(wid * (_BPW * _D), _BPW * _D)])


def _sc_entry(uidx_hbm, iidx_hbm, utT_hbm, itT_hbm, out_hbm,
              uidx_v, iidx_v, prod_v, *bufs_and_sems):
    _sc_body(uidx_hbm, iidx_hbm, utT_hbm, itT_hbm, out_hbm,
             uidx_v, iidx_v, prod_v, bufs_and_sems)


_sc_gather_prod = pl.kernel(
    _sc_entry,
    out_type=jax.ShapeDtypeStruct((_B * _D,), jnp.float32),
    mesh=plsc.VectorSubcoreMesh(core_axis_name="c", subcore_axis_name="s"),
    scratch_types=(
        [pltpu.VMEM((_BPW + 16,), jnp.int32),
         pltpu.VMEM((_BPW + 16,), jnp.int32),
         pltpu.VMEM((_BPW * _D,), jnp.float32)]
        + [pltpu.VMEM((_D, 128), jnp.float32)] * (2 * _NBUF)
        + [pltpu.SemaphoreType.DMA] * (2 * _NBUF)
    ),
    compiler_params=pltpu.CompilerParams(needs_layout_passes=False),
    name="sc_gather_prod",
)


def _tc_body(p_ref, wt4_ref, bt4_ref, wo4_ref, bo_ref, out_ref):
    h = jnp.dot(p_ref[...], wt4_ref[...], preferred_element_type=jnp.float32)
    h = jnp.maximum(h + bt4_ref[...], 0.0)
    out_ref[...] = jnp.dot(h, wo4_ref[...],
                           preferred_element_type=jnp.float32) + bo_ref[0, 0]


_tc_mlp = pl.pallas_call(
    _tc_body,
    out_shape=jax.ShapeDtypeStruct((_B // 4, 4), jnp.float32),
    name="tc_mlp",
)


def kernel(user_idx, item_idx, user_table, item_table, W_t, b_t, W_o, b_o):
    prod_flat = _sc_gather_prod(user_idx.astype(jnp.int32),
                                item_idx.astype(jnp.int32),
                                user_table.T, item_table.T)
    prod4 = prod_flat.reshape(_B // 4, 4 * _D)
    eye4 = jnp.eye(4, dtype=jnp.float32)
    wt4 = jnp.kron(eye4, W_t)
    wo4 = jnp.kron(eye4, W_o)
    bt4 = jnp.tile(b_t, 4).reshape(1, 4 * _D)
    pred = _tc_mlp(prod4, wt4, bt4, wo4, b_o.reshape(1, 1))
    return pred.reshape(_B)
